# baseline (speedup 1.0000x reference)
import functools

import jax
import jax.numpy as jnp
from jax import lax
from jax.experimental import pallas as pl
from jax.experimental.pallas import tpu as pltpu
from jax.experimental.pallas import tpu_sc as plsc

N = 10000
E = 320000
D = 128
NC = 2
NS = 16
NW = NC * NS
EPW = E // NW
C = 80
NCHUNK = EPW // C
G = 25
NG = NCHUNK // G
CP = 40
EPWP = 10240
NCP = EPWP // CP
NBUF = 4
QC = 64
NPAD = N + NW
RPT = 624
RREM = N - RPT * NS

ROWBLK = 1000
GRID = N // ROWBLK


def _hist_body(dstr, zrow, out, dst_v, ones_v, hist, sem):
    cid = lax.axis_index("c")
    sid = lax.axis_index("s")
    wid = cid * NS + sid
    pltpu.sync_copy(dstr.at[wid], dst_v)

    @pl.loop(0, C, step=16)
    def _(k):
        ones_v[pl.ds(k, 16)] = jnp.full((16,), 1.0, jnp.float32)

    @pl.when(sid == 0)
    def _():
        pltpu.async_copy(zrow, hist, sem).wait()

    plsc.subcore_barrier()

    @pl.loop(0, NG)
    def _(gi):
        @pl.loop(0, G)
        def _(j):
            pltpu.sync_copy(ones_v, hist.at[dst_v.at[gi, j]], add=True)

    plsc.subcore_barrier()

    @pl.when(sid == 0)
    def _():
        pltpu.sync_copy(hist, out.at[cid])


def _scatter_body(g, srcr, dstr, zblk, out, src_v, dst_v, bufs, acc,
                  semg, sems):
    cid = lax.axis_index("c")
    sid = lax.axis_index("s")
    wid = cid * NS + sid
    r0 = sid * RPT
    pltpu.sync_copy(srcr.at[wid], src_v)
    pltpu.sync_copy(dstr.at[wid * NBUF], dst_v)
    pltpu.sync_copy(zblk.at[pl.ds(r0, RPT)], acc.at[pl.ds(r0, RPT)])

    @pl.when(sid == NS - 1)
    def _():
        pltpu.sync_copy(zblk.at[pl.ds(RPT * NS, NPAD - RPT * NS)],
                        acc.at[pl.ds(RPT * NS, NPAD - RPT * NS)])

    plsc.subcore_barrier()

    def gidx(j):
        return g.at[src_v.at[pl.ds(j * CP, CP)]]

    for b in range(NBUF):
        pltpu.async_copy(gidx(b), bufs[b], semg[b])

    def quad(j, dj, tail):
        for b in range(NBUF):
            pltpu.make_async_copy(gidx(j + b), bufs[b], semg[b]).wait()
            pltpu.async_copy(bufs[b], acc.at[dst_v.at[dj + b]], sems[b],
                             add=True)
        for b in range(NBUF):
            pltpu.make_async_copy(bufs[b], acc.at[dst_v.at[dj + b]],
                                  sems[b]).wait()
            if not tail:
                pltpu.async_copy(gidx(j + NBUF + b), bufs[b], semg[b])

    for q in range(NCP // QC):
        if q > 0:
            pltpu.sync_copy(dstr.at[wid * NBUF + q], dst_v)
        last = q == NCP // QC - 1
        hi = QC - NBUF if last else QC

        @pl.loop(0, hi, step=NBUF)
        def _(j):
            quad(q * QC + j, j, False)

        if last:
            quad(NCP - NBUF, QC - NBUF, True)

    plsc.subcore_barrier()
    pltpu.sync_copy(acc.at[pl.ds(r0, RPT)], out.at[cid, pl.ds(r0, RPT)])

    @pl.when(sid == NS - 1)
    def _():
        pltpu.sync_copy(acc.at[pl.ds(RPT * NS, RREM)],
                        out.at[cid, pl.ds(RPT * NS, RREM)])


def _sc_hist(dstr, zrow):
    mesh = plsc.VectorSubcoreMesh(core_axis_name="c", subcore_axis_name="s")
    f = functools.partial(
        pl.kernel,
        out_type=jax.ShapeDtypeStruct((NC, N), jnp.float32),
        mesh=mesh,
        scratch_types=[
            pltpu.VMEM((NG, G, C), jnp.int32),
            pltpu.VMEM((C,), jnp.float32),
            pltpu.VMEM_SHARED((N,), jnp.float32),
            pltpu.SemaphoreType.DMA,
        ],
    )(_hist_body)
    return f(dstr, zrow)


def _sc_scatter(g, srcr, dstr, zblk):
    mesh = plsc.VectorSubcoreMesh(core_axis_name="c", subcore_axis_name="s")
    def body(g_, srcr_, dstr_, zblk_, out_, src_v, dst_v,
             b0, b1, b2, b3, acc, sg0, sg1, sg2, sg3, ss0, ss1, ss2, ss3):
        _scatter_body(g_, srcr_, dstr_, zblk_, out_, src_v, dst_v,
                      [b0, b1, b2, b3], acc,
                      [sg0, sg1, sg2, sg3], [ss0, ss1, ss2, ss3])

    f = functools.partial(
        pl.kernel,
        out_type=jax.ShapeDtypeStruct((NC, N, D), jnp.float32),
        mesh=mesh,
        scratch_types=(
            [pltpu.VMEM((EPWP,), jnp.int32),
             pltpu.VMEM((QC, CP), jnp.int32)]
            + [pltpu.VMEM((CP, D), jnp.float32) for _ in range(NBUF)]
            + [pltpu.VMEM_SHARED((NPAD, D), jnp.float32)]
            + [pltpu.SemaphoreType.DMA for _ in range(2 * NBUF)]
        ),
    )(body)
    return f(g, srcr, dstr, zblk)


def _mm1_body(x_ref, w_ref, deg_ref, g_ref, dinv_ref):
    dinv = lax.rsqrt(deg_ref[...])
    h = lax.dot_general(x_ref[...], w_ref[...], (((1,), (0,)), ((), ())),
                        precision=lax.Precision.HIGHEST)
    g_ref[...] = dinv * h
    dinv_ref[...] = dinv


def _mm2_body(s_ref, g_ref, dinv_ref, b_ref, w_ref, g2_ref):
    dinv = dinv_ref[...]
    pre = dinv * (s_ref[0] + s_ref[1] + g_ref[...]) + b_ref[...]
    h = jnp.maximum(pre, 0.0)
    h2 = lax.dot_general(h, w_ref[...], (((1,), (0,)), ((), ())),
                         precision=lax.Precision.HIGHEST)
    g2_ref[...] = dinv * h2


def _out_body(s_ref, g_ref, dinv_ref, b_ref, o_ref):
    z = dinv_ref[...] * (s_ref[0] + s_ref[1] + g_ref[...]) + b_ref[...]
    m = jnp.max(z, axis=1, keepdims=True)
    lse = jnp.log(jnp.sum(jnp.exp(z - m), axis=1, keepdims=True)) + m
    o_ref[...] = z - lse


_ROW = pl.BlockSpec((ROWBLK, D), lambda i: (i, 0))
_ROW1 = pl.BlockSpec((ROWBLK, 1), lambda i: (i, 0))
_FULL = pl.BlockSpec((D, D), lambda i: (0, 0))
_BIAS = pl.BlockSpec((1, D), lambda i: (0, 0))
_PAIR = pl.BlockSpec((NC, ROWBLK, D), lambda i: (0, i, 0))


def _tc_mm1(x, w, deg):
    return pl.pallas_call(
        _mm1_body,
        grid=(GRID,),
        in_specs=[_ROW, _FULL, _ROW1],
        out_specs=[_ROW, _ROW1],
        out_shape=[jax.ShapeDtypeStruct((N, D), jnp.float32),
                   jax.ShapeDtypeStruct((N, 1), jnp.float32)],
    )(x, w, deg)


def _tc_mm2(s, g, dinv, b, w):
    return pl.pallas_call(
        _mm2_body,
        grid=(GRID,),
        in_specs=[_PAIR, _ROW, _ROW1, _BIAS, _FULL],
        out_specs=_ROW,
        out_shape=jax.ShapeDtypeStruct((N, D), jnp.float32),
    )(s, g, dinv, b, w)


def _tc_out(s, g, dinv, b):
    return pl.pallas_call(
        _out_body,
        grid=(GRID,),
        in_specs=[_PAIR, _ROW, _ROW1, _BIAS],
        out_specs=_ROW,
        out_shape=jax.ShapeDtypeStruct((N, D), jnp.float32),
    )(s, g, dinv, b)


def kernel(x, edge_index, W1, b1, W2, b2):
    pad = jnp.broadcast_to(
        (N + jnp.arange(NW, dtype=jnp.int32))[:, None], (NW, EPWP - EPW))
    src = jnp.concatenate(
        [edge_index[0].reshape(NW, EPW),
         jnp.zeros((NW, EPWP - EPW), jnp.int32)], axis=1)
    dst2 = jnp.concatenate(
        [edge_index[1].reshape(NW, EPW), pad],
        axis=1).reshape(NW * NBUF, QC, CP)
    dst = edge_index[1].reshape(NW, NG, G, C)
    zrow = jnp.zeros((N,), jnp.float32)
    zblk = jnp.zeros((NPAD, D), jnp.float32)

    hist = _sc_hist(dst, zrow)
    deg = (hist[0] + hist[1] + 1.0).reshape(N, 1)

    g1, dinv = _tc_mm1(x, W1, deg)
    s1 = _sc_scatter(g1, src, dst2, zblk)
    g2 = _tc_mm2(s1, g1, dinv, b1.reshape(1, D), W2)
    s2 = _sc_scatter(g2, src, dst2, zblk)
    return _tc_out(s2, g2, dinv, b2.reshape(1, D))


# speedup vs baseline: 2.6331x; 2.6331x over previous
import functools

import jax
import jax.numpy as jnp
from jax import lax
from jax.experimental import pallas as pl
from jax.experimental.pallas import tpu as pltpu
from jax.experimental.pallas import tpu_sc as plsc

N = 10000
E = 320000
D = 128
NC = 2
NS = 16
NW = NC * NS
EPW = E // NW
C = 80
NCHUNK = EPW // C
G = 25
NG = NCHUNK // G
CP = 80
EPWP = 10000
NCP = EPWP // CP
NBUF = 3
HC = 63
NPAD = N + NW
RPT = 624
RREM = N - RPT * NS

ROWBLK = 1000
GRID = N // ROWBLK


def _hist_body(dstr, zrow, out, dst_v, ones_v, hist, sem):
    cid = lax.axis_index("c")
    sid = lax.axis_index("s")
    wid = cid * NS + sid
    pltpu.sync_copy(dstr.at[wid], dst_v)

    @pl.loop(0, C, step=16)
    def _(k):
        ones_v[pl.ds(k, 16)] = jnp.full((16,), 1.0, jnp.float32)

    @pl.when(sid == 0)
    def _():
        pltpu.async_copy(zrow, hist, sem).wait()

    plsc.subcore_barrier()

    @pl.loop(0, NG)
    def _(gi):
        @pl.loop(0, G)
        def _(j):
            pltpu.sync_copy(ones_v, hist.at[dst_v.at[gi, j]], add=True)

    plsc.subcore_barrier()

    @pl.when(sid == 0)
    def _():
        pltpu.sync_copy(hist, out.at[cid])


def _scatter_body(g, srcr, dstr, zblk, out, src_v, dst_v, bufs, acc,
                  semg, sems):
    cid = lax.axis_index("c")
    sid = lax.axis_index("s")
    wid = cid * NS + sid
    r0 = sid * RPT
    pltpu.sync_copy(srcr.at[wid], src_v)
    pltpu.sync_copy(dstr.at[wid * 2], dst_v)
    pltpu.sync_copy(zblk.at[pl.ds(r0, RPT)], acc.at[pl.ds(r0, RPT)])

    @pl.when(sid == NS - 1)
    def _():
        pltpu.sync_copy(zblk.at[pl.ds(RPT * NS, NPAD - RPT * NS)],
                        acc.at[pl.ds(RPT * NS, NPAD - RPT * NS)])

    plsc.subcore_barrier()

    def gidx(j):
        return g.at[src_v.at[pl.ds(j * CP, CP)]]

    for b in range(NBUF):
        pltpu.async_copy(gidx(b), bufs[b], semg[b])

    def triple(j, dj):
        for b in range(NBUF):
            pltpu.make_async_copy(gidx(j + b), bufs[b], semg[b]).wait()
            pltpu.async_copy(bufs[b], acc.at[dst_v.at[dj + b]], sems[b],
                             add=True)
        for b in range(NBUF):
            pltpu.make_async_copy(bufs[b], acc.at[dst_v.at[dj + b]],
                                  sems[b]).wait()

            @pl.when(j + NBUF + b < NCP)
            def _():
                pltpu.async_copy(gidx(j + NBUF + b), bufs[b], semg[b])

    @pl.loop(0, HC, step=NBUF)
    def _(j):
        triple(j, j)

    pltpu.sync_copy(dstr.at[wid * 2 + 1], dst_v)

    @pl.loop(HC, HC + 60, step=NBUF)
    def _(j):
        triple(j, j - HC)

    pltpu.make_async_copy(gidx(NCP - 2), bufs[0], semg[0]).wait()
    pltpu.sync_copy(bufs[0], acc.at[dst_v.at[60]], add=True)
    pltpu.make_async_copy(gidx(NCP - 1), bufs[1], semg[1]).wait()
    pltpu.sync_copy(bufs[1], acc.at[dst_v.at[61]], add=True)

    plsc.subcore_barrier()
    pltpu.sync_copy(acc.at[pl.ds(r0, RPT)], out.at[cid, pl.ds(r0, RPT)])

    @pl.when(sid == NS - 1)
    def _():
        pltpu.sync_copy(acc.at[pl.ds(RPT * NS, RREM)],
                        out.at[cid, pl.ds(RPT * NS, RREM)])


def _sc_hist(dstr, zrow):
    mesh = plsc.VectorSubcoreMesh(core_axis_name="c", subcore_axis_name="s")
    f = functools.partial(
        pl.kernel,
        out_type=jax.ShapeDtypeStruct((NC, N), jnp.float32),
        mesh=mesh,
        scratch_types=[
            pltpu.VMEM((NG, G, C), jnp.int32),
            pltpu.VMEM((C,), jnp.float32),
            pltpu.VMEM_SHARED((N,), jnp.float32),
            pltpu.SemaphoreType.DMA,
        ],
    )(_hist_body)
    return f(dstr, zrow)


def _sc_scatter(g, srcr, dstr, zblk):
    mesh = plsc.VectorSubcoreMesh(core_axis_name="c", subcore_axis_name="s")
    def body(g_, srcr_, dstr_, zblk_, out_, src_v, dst_v, *rest):
        bufs = list(rest[:NBUF])
        acc = rest[NBUF]
        semg = list(rest[NBUF + 1:2 * NBUF + 1])
        sems = list(rest[2 * NBUF + 1:])
        _scatter_body(g_, srcr_, dstr_, zblk_, out_, src_v, dst_v,
                      bufs, acc, semg, sems)

    f = functools.partial(
        pl.kernel,
        out_type=jax.ShapeDtypeStruct((NC, N, D), jnp.float32),
        mesh=mesh,
        scratch_types=(
            [pltpu.VMEM((EPWP,), jnp.int32),
             pltpu.VMEM((HC, CP), jnp.int32)]
            + [pltpu.VMEM((CP, D), jnp.float32) for _ in range(NBUF)]
            + [pltpu.VMEM_SHARED((NPAD, D), jnp.float32)]
            + [pltpu.SemaphoreType.DMA for _ in range(2 * NBUF)]
        ),
    )(body)
    return f(g, srcr, dstr, zblk)


def _mm1_body(x_ref, w_ref, deg_ref, g_ref, dinv_ref):
    dinv = lax.rsqrt(deg_ref[...])
    h = lax.dot_general(x_ref[...], w_ref[...], (((1,), (0,)), ((), ())),
                        precision=lax.Precision.HIGHEST)
    g_ref[...] = dinv * h
    dinv_ref[...] = dinv


def _mm2_body(s_ref, g_ref, dinv_ref, b_ref, w_ref, g2_ref):
    dinv = dinv_ref[...]
    pre = dinv * (s_ref[0] + s_ref[1] + g_ref[...]) + b_ref[...]
    h = jnp.maximum(pre, 0.0)
    h2 = lax.dot_general(h, w_ref[...], (((1,), (0,)), ((), ())),
                         precision=lax.Precision.HIGHEST)
    g2_ref[...] = dinv * h2


def _out_body(s_ref, g_ref, dinv_ref, b_ref, o_ref):
    z = dinv_ref[...] * (s_ref[0] + s_ref[1] + g_ref[...]) + b_ref[...]
    m = jnp.max(z, axis=1, keepdims=True)
    lse = jnp.log(jnp.sum(jnp.exp(z - m), axis=1, keepdims=True)) + m
    o_ref[...] = z - lse


_ROW = pl.BlockSpec((ROWBLK, D), lambda i: (i, 0))
_ROW1 = pl.BlockSpec((ROWBLK, 1), lambda i: (i, 0))
_FULL = pl.BlockSpec((D, D), lambda i: (0, 0))
_BIAS = pl.BlockSpec((1, D), lambda i: (0, 0))
_PAIR = pl.BlockSpec((NC, ROWBLK, D), lambda i: (0, i, 0))


def _tc_mm1(x, w, deg):
    return pl.pallas_call(
        _mm1_body,
        grid=(GRID,),
        in_specs=[_ROW, _FULL, _ROW1],
        out_specs=[_ROW, _ROW1],
        out_shape=[jax.ShapeDtypeStruct((N, D), jnp.float32),
                   jax.ShapeDtypeStruct((N, 1), jnp.float32)],
    )(x, w, deg)


def _tc_mm2(s, g, dinv, b, w):
    return pl.pallas_call(
        _mm2_body,
        grid=(GRID,),
        in_specs=[_PAIR, _ROW, _ROW1, _BIAS, _FULL],
        out_specs=_ROW,
        out_shape=jax.ShapeDtypeStruct((N, D), jnp.float32),
    )(s, g, dinv, b, w)


def _tc_out(s, g, dinv, b):
    return pl.pallas_call(
        _out_body,
        grid=(GRID,),
        in_specs=[_PAIR, _ROW, _ROW1, _BIAS],
        out_specs=_ROW,
        out_shape=jax.ShapeDtypeStruct((N, D), jnp.float32),
    )(s, g, dinv, b)


def kernel(x, edge_index, W1, b1, W2, b2):
    src = edge_index[0].reshape(NW, EPW)
    dst_r = edge_index[1].reshape(NW, NCP, CP)
    dst2 = jnp.concatenate(
        [dst_r[:, None, :HC],
         jnp.concatenate([dst_r[:, None, HC:],
                          jnp.zeros((NW, 1, 1, CP), jnp.int32)], axis=2)],
        axis=1).reshape(NW * 2, HC, CP)
    dst = edge_index[1].reshape(NW, NG, G, C)
    zrow = jnp.zeros((N,), jnp.float32)
    zblk = jnp.zeros((NPAD, D), jnp.float32)

    hist = _sc_hist(dst, zrow)
    deg = (hist[0] + hist[1] + 1.0).reshape(N, 1)

    g1, dinv = _tc_mm1(x, W1, deg)
    s1 = _sc_scatter(g1, src, dst2, zblk)
    g2 = _tc_mm2(s1, g1, dinv, b1.reshape(1, D), W2)
    s2 = _sc_scatter(g2, src, dst2, zblk)
    return _tc_out(s2, g2, dinv, b2.reshape(1, D))
